# Initial kernel scaffold; baseline (speedup 1.0000x reference)
import jax
import jax.numpy as jnp
from jax.experimental import pallas as pl


def kernel(x, edge_index, batch, W1, b1, W2, b2):
    raise NotImplementedError("write your pallas kernel here")


# speedup vs baseline: 43.1655x; 43.1655x over previous
import functools

import jax
import jax.numpy as jnp
from jax import lax
from jax.experimental import pallas as pl
from jax.experimental.pallas import tpu as pltpu
from jax.experimental.pallas import tpu_sc as plsc

N = 10000
NP = 10240
E = 320000
D = 128
H16 = 16
C = 10
G = 64

NC = 2
NS = 16
NW = NC * NS
BLK = 128
NBLK = 80
EP = NW * NBLK * BLK
RPT = NP // NS
NBUF = 4
R = 1024
NR = NP // R

_mesh = plsc.VectorSubcoreMesh(core_axis_name="c", subcore_axis_name="s")
_sc_params = pltpu.CompilerParams(use_tc_tiling_on_sc=False)
_sc_params_nl = pltpu.CompilerParams(use_tc_tiling_on_sc=False,
                                     needs_layout_passes=False)


@functools.partial(
    pl.kernel,
    out_type=jax.ShapeDtypeStruct((NC, NP), jnp.float32),
    mesh=_mesh,
    scratch_types=[
        pltpu.VMEM((NBLK, BLK), jnp.int32),
        pltpu.VMEM((NP,), jnp.float32),
        pltpu.VMEM((RPT,), jnp.float32),
        pltpu.VMEM((RPT,), jnp.float32),
        pltpu.VMEM_SHARED((NS, NP), jnp.float32),
        pltpu.SemaphoreType.DMA,
    ],
    compiler_params=_sc_params_nl,
)
def _sc_deg(dst_hbm, out_hbm, didx, hist, loc, tmp, stage, sem):
    c = lax.axis_index("c")
    s = lax.axis_index("s")
    wid = c * NS + s
    pltpu.sync_copy(dst_hbm.at[pl.ds(wid * NBLK, NBLK)], didx)
    ones = jnp.ones((16,), jnp.float32)
    zero = jnp.zeros((16,), jnp.float32)

    def zloop(i, carry):
        hist[pl.ds(i * 16, 16)] = zero
        return carry
    lax.fori_loop(0, NP // 16, zloop, 0)

    def body(i, carry):
        j = i // 8
        k = (i % 8) * 16
        idx = didx[j, pl.ds(k, 16)]
        plsc.addupdate_scatter(hist, [idx], ones)
        return carry
    lax.fori_loop(0, NBLK * BLK // 16, body, 0)

    pltpu.sync_copy(hist, stage.at[s])
    plsc.subcore_barrier()

    def zl2(i, carry):
        loc[pl.ds(i * 16, 16)] = zero
        return carry
    lax.fori_loop(0, RPT // 16, zl2, 0)

    def red(t, carry):
        pltpu.sync_copy(stage.at[t, pl.ds(s * RPT, RPT)], tmp)

        def add(i, carry2):
            loc[pl.ds(i * 16, 16)] = (loc[pl.ds(i * 16, 16)]
                                      + tmp[pl.ds(i * 16, 16)])
            return carry2
        lax.fori_loop(0, RPT // 16, add, 0)
        return carry
    lax.fori_loop(0, NS, red, 0)
    pltpu.sync_copy(loc, out_hbm.at[c, pl.ds(s * RPT, RPT)])


@functools.partial(
    pl.kernel,
    out_type=jax.ShapeDtypeStruct((NC, NP, H16), jnp.float32),
    mesh=_mesh,
    scratch_types=[
        pltpu.VMEM((NBLK, BLK), jnp.int32),
        pltpu.VMEM((NBLK, BLK), jnp.int32),
        pltpu.VMEM((NBUF, BLK, H16), jnp.float32),
        pltpu.VMEM_SHARED((NP, H16), jnp.float32),
        pltpu.SemaphoreType.DMA,
    ],
    compiler_params=_sc_params,
)
def _sc_agg(g_hbm, src_hbm, dst_hbm, z16_hbm, out_hbm,
            sidx, didx, rows, acc, sem):
    c = lax.axis_index("c")
    s = lax.axis_index("s")
    wid = c * NS + s
    pltpu.sync_copy(z16_hbm, acc.at[pl.ds(s * RPT, RPT)])
    pltpu.sync_copy(src_hbm.at[pl.ds(wid * NBLK, NBLK)], sidx)
    pltpu.sync_copy(dst_hbm.at[pl.ds(wid * NBLK, NBLK)], didx)
    plsc.subcore_barrier()

    def grp(gi, carry):
        cps = []
        for b in range(NBUF):
            j = gi * NBUF + b
            cps.append(pltpu.async_copy(g_hbm.at[sidx.at[j]], rows.at[b], sem))
        for b in range(NBUF):
            cps[b].wait()
        for b in range(NBUF):
            j = gi * NBUF + b
            pltpu.sync_copy(rows.at[b], acc.at[didx.at[j]], add=True)
        return carry

    lax.fori_loop(0, NBLK // NBUF, grp, 0)
    plsc.subcore_barrier()
    pltpu.sync_copy(acc.at[pl.ds(s * RPT, RPT)],
                    out_hbm.at[c, pl.ds(s * RPT, RPT)])


def _tc1_body(x_ref, w_ref, d0_ref, d1_ref, g_ref, h_ref, dinv_ref):
    i = pl.program_id(0)
    h = jnp.dot(x_ref[...], w_ref[...], preferred_element_type=jnp.float32)
    deg = d0_ref[...] + d1_ref[...] + 1.0
    dinv = lax.rsqrt(jnp.maximum(deg, 1.0))
    rows = lax.broadcasted_iota(jnp.int32, (R, 1), 0) + i * R
    dinv = jnp.where(rows < N, dinv, 0.0)
    h_ref[...] = h
    g_ref[...] = dinv * h
    dinv_ref[...] = dinv


def _tc1(x_pad, W1, d0, d1):
    return pl.pallas_call(
        _tc1_body,
        grid=(NR,),
        in_specs=[
            pl.BlockSpec((R, D), lambda i: (i, 0)),
            pl.BlockSpec((D, H16), lambda i: (0, 0)),
            pl.BlockSpec((R, 1), lambda i: (i, 0)),
            pl.BlockSpec((R, 1), lambda i: (i, 0)),
        ],
        out_specs=[
            pl.BlockSpec((R, H16), lambda i: (i, 0)),
            pl.BlockSpec((R, H16), lambda i: (i, 0)),
            pl.BlockSpec((R, 1), lambda i: (i, 0)),
        ],
        out_shape=[
            jax.ShapeDtypeStruct((NP, H16), jnp.float32),
            jax.ShapeDtypeStruct((NP, H16), jnp.float32),
            jax.ShapeDtypeStruct((NP, 1), jnp.float32),
        ],
    )(x_pad, W1, d0, d1)


def _tc2_body(p0_ref, p1_ref, h1_ref, dinv_ref, b1_ref, w2_ref,
              g2_ref, h2_ref):
    i = pl.program_id(0)
    dinv = dinv_ref[...]
    out1 = dinv * (p0_ref[...] + p1_ref[...]) \
        + dinv * dinv * h1_ref[...] + b1_ref[...]
    out1 = jnp.maximum(out1, 0.0)
    rows = lax.broadcasted_iota(jnp.int32, (R, 1), 0) + i * R
    out1 = jnp.where(rows < N, out1, 0.0)
    h2 = jnp.dot(out1, w2_ref[...], preferred_element_type=jnp.float32)
    h2_ref[...] = h2
    g2_ref[...] = dinv * h2


def _tc2(p0, p1, h1, dinv, b1r, W2p):
    return pl.pallas_call(
        _tc2_body,
        grid=(NR,),
        in_specs=[
            pl.BlockSpec((R, H16), lambda i: (i, 0)),
            pl.BlockSpec((R, H16), lambda i: (i, 0)),
            pl.BlockSpec((R, H16), lambda i: (i, 0)),
            pl.BlockSpec((R, 1), lambda i: (i, 0)),
            pl.BlockSpec((1, H16), lambda i: (0, 0)),
            pl.BlockSpec((H16, H16), lambda i: (0, 0)),
        ],
        out_specs=[
            pl.BlockSpec((R, H16), lambda i: (i, 0)),
            pl.BlockSpec((R, H16), lambda i: (i, 0)),
        ],
        out_shape=[
            jax.ShapeDtypeStruct((NP, H16), jnp.float32),
            jax.ShapeDtypeStruct((NP, H16), jnp.float32),
        ],
    )(p0, p1, h1, dinv, b1r, W2p)


def _tc3_body(q0_ref, q1_ref, h2_ref, dinv_ref, b2_ref, batch_ref,
              out_ref, acc):
    i = pl.program_id(0)

    @pl.when(i == 0)
    def _init():
        acc[...] = jnp.zeros((G, H16), jnp.float32)

    dinv = dinv_ref[...]
    out2 = dinv * (q0_ref[...] + q1_ref[...]) \
        + dinv * dinv * h2_ref[...] + b2_ref[...]
    rows = lax.broadcasted_iota(jnp.int32, (R, 1), 0) + i * R
    m = (rows < N).astype(jnp.float32)
    out2 = out2 * m
    colmask = lax.broadcasted_iota(jnp.int32, (R, H16), 1) == (H16 - 1)
    out2 = jnp.where(colmask, m, out2)
    bb = batch_ref[0, 0, :]
    onehot = (bb[None, :] ==
              lax.broadcasted_iota(jnp.int32, (G, R), 0)).astype(jnp.float32)
    acc[...] += jnp.dot(onehot, out2, preferred_element_type=jnp.float32)

    @pl.when(i == NR - 1)
    def _fin():
        a = acc[...]
        cnt = jnp.maximum(a[:, H16 - 1:H16], 1.0)
        out_ref[...] = jax.nn.sigmoid(a / cnt)


def _tc3(q0, q1, h2, dinv, b2r, batch3):
    return pl.pallas_call(
        _tc3_body,
        grid=(NR,),
        in_specs=[
            pl.BlockSpec((R, H16), lambda i: (i, 0)),
            pl.BlockSpec((R, H16), lambda i: (i, 0)),
            pl.BlockSpec((R, H16), lambda i: (i, 0)),
            pl.BlockSpec((R, 1), lambda i: (i, 0)),
            pl.BlockSpec((1, H16), lambda i: (0, 0)),
            pl.BlockSpec((1, 1, R), lambda i: (i, 0, 0)),
        ],
        out_specs=pl.BlockSpec((G, H16), lambda i: (0, 0)),
        out_shape=jax.ShapeDtypeStruct((G, H16), jnp.float32),
        scratch_shapes=[pltpu.VMEM((G, H16), jnp.float32)],
    )(q0, q1, h2, dinv, b2r, batch3)


def kernel(x, edge_index, batch, W1, b1, W2, b2):
    f32 = jnp.float32
    src = edge_index[0]
    dst = edge_index[1]
    pad = N + (jnp.arange(EP - E, dtype=jnp.int32) % (NP - N))
    src_r = jnp.concatenate([src, pad]).reshape(NW * NBLK, BLK)
    dst_r = jnp.concatenate([dst, pad]).reshape(NW * NBLK, BLK)
    x_pad = jnp.zeros((NP, D), f32).at[:N].set(x)
    z16 = jnp.zeros((RPT, H16), f32)
    b1r = b1.reshape(1, H16)
    W2p = jnp.zeros((H16, H16), f32).at[:, :C].set(W2)
    b2r = jnp.zeros((1, H16), f32).at[0, :C].set(b2)
    batch3 = jnp.concatenate(
        [batch, jnp.zeros((NP - N,), jnp.int32)]).reshape(NR, 1, R)

    degp = _sc_deg(dst_r)
    g1, h1, dinv = _tc1(x_pad, W1,
                        degp[0].reshape(NP, 1), degp[1].reshape(NP, 1))
    p = _sc_agg(g1, src_r, dst_r, z16)
    g2, h2 = _tc2(p[0], p[1], h1, dinv, b1r, W2p)
    q = _sc_agg(g2, src_r, dst_r, z16)
    out16 = _tc3(q[0], q[1], h2, dinv, b2r, batch3)
    return out16[:, :C]


# speedup vs baseline: 47.0945x; 1.0910x over previous
import functools

import jax
import jax.numpy as jnp
import numpy as np
from jax import lax
from jax.experimental import pallas as pl
from jax.experimental.pallas import tpu as pltpu
from jax.experimental.pallas import tpu_sc as plsc

N = 10000
NP = 10240
E = 320000
D = 128
H16 = 16
C = 10
G = 64

NC = 2
NS = 16
NW = NC * NS
BLK = 128
NBLK = 80
EP = NW * NBLK * BLK
RPT = NP // NS
NBUF = 4
R = 1024
NR = NP // R

_PAD_IDX = np.asarray(N + np.arange(EP - E) % (NP - N), np.int32)
_PAD2 = np.broadcast_to(_PAD_IDX, (2, EP - E))

_mesh = plsc.VectorSubcoreMesh(core_axis_name="c", subcore_axis_name="s")
_sc_params = pltpu.CompilerParams(use_tc_tiling_on_sc=False)
_sc_params_nl = pltpu.CompilerParams(use_tc_tiling_on_sc=False,
                                     needs_layout_passes=False)


@functools.partial(
    pl.kernel,
    out_type=jax.ShapeDtypeStruct((NC, NP), jnp.float32),
    mesh=_mesh,
    scratch_types=[
        pltpu.VMEM((NBLK, BLK), jnp.int32),
        pltpu.VMEM((NP,), jnp.float32),
        pltpu.VMEM((RPT,), jnp.float32),
        pltpu.VMEM((RPT,), jnp.float32),
        pltpu.VMEM_SHARED((NS, NP), jnp.float32),
        pltpu.SemaphoreType.DMA,
    ],
    compiler_params=_sc_params_nl,
)
def _sc_deg(ei_hbm, out_hbm, didx, hist, loc, tmp, stage, sem):
    c = lax.axis_index("c")
    s = lax.axis_index("s")
    wid = c * NS + s
    pltpu.sync_copy(ei_hbm.at[1, pl.ds(wid * NBLK, NBLK)], didx)
    ones = jnp.ones((16,), jnp.float32)
    zero = jnp.zeros((16,), jnp.float32)

    def zloop(i, carry):
        hist[pl.ds(i * 16, 16)] = zero
        return carry
    lax.fori_loop(0, NP // 16, zloop, 0)

    def body(i, carry):
        j = i // 8
        k = (i % 8) * 16
        idx = didx[j, pl.ds(k, 16)]
        plsc.addupdate_scatter(hist, [idx], ones)
        return carry
    lax.fori_loop(0, NBLK * BLK // 16, body, 0)

    pltpu.sync_copy(hist, stage.at[s])
    plsc.subcore_barrier()

    def zl2(i, carry):
        loc[pl.ds(i * 16, 16)] = zero
        return carry
    lax.fori_loop(0, RPT // 16, zl2, 0)

    def red(t, carry):
        pltpu.sync_copy(stage.at[t, pl.ds(s * RPT, RPT)], tmp)

        def add(i, carry2):
            loc[pl.ds(i * 16, 16)] = (loc[pl.ds(i * 16, 16)]
                                      + tmp[pl.ds(i * 16, 16)])
            return carry2
        lax.fori_loop(0, RPT // 16, add, 0)
        return carry
    lax.fori_loop(0, NS, red, 0)
    pltpu.sync_copy(loc, out_hbm.at[c, pl.ds(s * RPT, RPT)])


@functools.partial(
    pl.kernel,
    out_type=jax.ShapeDtypeStruct((NC, NP, H16), jnp.float32),
    mesh=_mesh,
    scratch_types=[
        pltpu.VMEM((NBLK, BLK), jnp.int32),
        pltpu.VMEM((NBLK, BLK), jnp.int32),
        pltpu.VMEM((NBUF, BLK, H16), jnp.float32),
        pltpu.VMEM_SHARED((NP, H16), jnp.float32),
        pltpu.SemaphoreType.DMA,
    ],
    compiler_params=_sc_params,
)
def _sc_agg(g_hbm, ei_hbm, z16_hbm, out_hbm, sidx, didx, rows, acc, sem):
    c = lax.axis_index("c")
    s = lax.axis_index("s")
    wid = c * NS + s
    pltpu.sync_copy(z16_hbm, acc.at[pl.ds(s * RPT, RPT)])
    pltpu.sync_copy(ei_hbm.at[0, pl.ds(wid * NBLK, NBLK)], sidx)
    pltpu.sync_copy(ei_hbm.at[1, pl.ds(wid * NBLK, NBLK)], didx)
    plsc.subcore_barrier()

    def grp(gi, carry):
        cps = []
        for b in range(NBUF):
            j = gi * NBUF + b
            cps.append(pltpu.async_copy(g_hbm.at[sidx.at[j]], rows.at[b], sem))
        for b in range(NBUF):
            cps[b].wait()
        for b in range(NBUF):
            j = gi * NBUF + b
            pltpu.sync_copy(rows.at[b], acc.at[didx.at[j]], add=True)
        return carry

    lax.fori_loop(0, NBLK // NBUF, grp, 0)
    plsc.subcore_barrier()
    pltpu.sync_copy(acc.at[pl.ds(s * RPT, RPT)],
                    out_hbm.at[c, pl.ds(s * RPT, RPT)])


def _tc1a_body(x_ref, w_ref, h_ref):
    h_ref[...] = jnp.dot(x_ref[...], w_ref[...],
                         preferred_element_type=jnp.float32)


def _tc1a(x_pad, W1):
    return pl.pallas_call(
        _tc1a_body,
        grid=(NR,),
        in_specs=[
            pl.BlockSpec((R, D), lambda i: (i, 0)),
            pl.BlockSpec((D, H16), lambda i: (0, 0)),
        ],
        out_specs=pl.BlockSpec((R, H16), lambda i: (i, 0)),
        out_shape=jax.ShapeDtypeStruct((NP, H16), jnp.float32),
    )(x_pad, W1)


def _tc1b_body(deg_ref, h_ref, g_ref, dinv_ref):
    i = pl.program_id(0)
    deg = deg_ref[0] + deg_ref[1] + 1.0
    dinv = lax.rsqrt(jnp.maximum(deg, 1.0))
    rows = lax.broadcasted_iota(jnp.int32, (R, 1), 0) + i * R
    dinv = jnp.where(rows < N, dinv, 0.0)
    g_ref[...] = dinv * h_ref[...]
    dinv_ref[...] = dinv


def _tc1b(degp, h1):
    return pl.pallas_call(
        _tc1b_body,
        grid=(NR,),
        in_specs=[
            pl.BlockSpec((NC, R, 1), lambda i: (0, i, 0)),
            pl.BlockSpec((R, H16), lambda i: (i, 0)),
        ],
        out_specs=[
            pl.BlockSpec((R, H16), lambda i: (i, 0)),
            pl.BlockSpec((R, 1), lambda i: (i, 0)),
        ],
        out_shape=[
            jax.ShapeDtypeStruct((NP, H16), jnp.float32),
            jax.ShapeDtypeStruct((NP, 1), jnp.float32),
        ],
    )(degp, h1)


def _tc2_body(p_ref, h1_ref, dinv_ref, b1_ref, w2_ref, g2_ref, h2_ref):
    i = pl.program_id(0)
    dinv = dinv_ref[...]
    out1 = dinv * (p_ref[0] + p_ref[1]) \
        + dinv * dinv * h1_ref[...] + b1_ref[...]
    out1 = jnp.maximum(out1, 0.0)
    rows = lax.broadcasted_iota(jnp.int32, (R, 1), 0) + i * R
    out1 = jnp.where(rows < N, out1, 0.0)
    h2 = jnp.dot(out1, w2_ref[...], preferred_element_type=jnp.float32)
    h2_ref[...] = h2
    g2_ref[...] = dinv * h2


def _tc2(p, h1, dinv, b1r, W2p):
    return pl.pallas_call(
        _tc2_body,
        grid=(NR,),
        in_specs=[
            pl.BlockSpec((NC, R, H16), lambda i: (0, i, 0)),
            pl.BlockSpec((R, H16), lambda i: (i, 0)),
            pl.BlockSpec((R, 1), lambda i: (i, 0)),
            pl.BlockSpec((1, H16), lambda i: (0, 0)),
            pl.BlockSpec((H16, H16), lambda i: (0, 0)),
        ],
        out_specs=[
            pl.BlockSpec((R, H16), lambda i: (i, 0)),
            pl.BlockSpec((R, H16), lambda i: (i, 0)),
        ],
        out_shape=[
            jax.ShapeDtypeStruct((NP, H16), jnp.float32),
            jax.ShapeDtypeStruct((NP, H16), jnp.float32),
        ],
    )(p, h1, dinv, b1r, W2p)


def _tc3_body(q_ref, h2_ref, dinv_ref, b2_ref, batch_ref, out_ref, acc):
    i = pl.program_id(0)

    @pl.when(i == 0)
    def _init():
        acc[...] = jnp.zeros((G, H16), jnp.float32)

    dinv = dinv_ref[...]
    out2 = dinv * (q_ref[0] + q_ref[1]) \
        + dinv * dinv * h2_ref[...] + b2_ref[...]
    rows = lax.broadcasted_iota(jnp.int32, (R, 1), 0) + i * R
    m = (rows < N).astype(jnp.float32)
    out2 = out2 * m
    colmask = lax.broadcasted_iota(jnp.int32, (R, H16), 1) == (H16 - 1)
    out2 = jnp.where(colmask, m, out2)
    bb = batch_ref[0, 0, :]
    onehot = (bb[None, :] ==
              lax.broadcasted_iota(jnp.int32, (G, R), 0)).astype(jnp.float32)
    acc[...] += jnp.dot(onehot, out2, preferred_element_type=jnp.float32)

    @pl.when(i == NR - 1)
    def _fin():
        a = acc[...]
        cnt = jnp.maximum(a[:, H16 - 1:H16], 1.0)
        out_ref[...] = jax.nn.sigmoid(a / cnt)


def _tc3(q, h2, dinv, b2r, batch3):
    return pl.pallas_call(
        _tc3_body,
        grid=(NR,),
        in_specs=[
            pl.BlockSpec((NC, R, H16), lambda i: (0, i, 0)),
            pl.BlockSpec((R, H16), lambda i: (i, 0)),
            pl.BlockSpec((R, 1), lambda i: (i, 0)),
            pl.BlockSpec((1, H16), lambda i: (0, 0)),
            pl.BlockSpec((1, 1, R), lambda i: (i, 0, 0)),
        ],
        out_specs=pl.BlockSpec((G, H16), lambda i: (0, 0)),
        out_shape=jax.ShapeDtypeStruct((G, H16), jnp.float32),
        scratch_shapes=[pltpu.VMEM((G, H16), jnp.float32)],
    )(q, h2, dinv, b2r, batch3)


def kernel(x, edge_index, batch, W1, b1, W2, b2):
    f32 = jnp.float32
    ei_r = jnp.concatenate(
        [edge_index, jnp.asarray(_PAD2)], axis=1).reshape(2, NW * NBLK, BLK)
    x_pad = jnp.zeros((NP, D), f32).at[:N].set(x)
    z16 = jnp.zeros((RPT, H16), f32)
    b1r = b1.reshape(1, H16)
    W2p = jnp.zeros((H16, H16), f32).at[:, :C].set(W2)
    b2r = jnp.zeros((1, H16), f32).at[0, :C].set(b2)
    batch3 = jnp.concatenate(
        [batch, jnp.zeros((NP - N,), jnp.int32)]).reshape(NR, 1, R)

    h1 = _tc1a(x_pad, W1)
    degp = _sc_deg(ei_r)
    g1, dinv = _tc1b(degp.reshape(NC, NP, 1), h1)
    p = _sc_agg(g1, ei_r, z16)
    g2, h2 = _tc2(p, h1, dinv, b1r, W2p)
    q = _sc_agg(g2, ei_r, z16)
    out16 = _tc3(q, h2, dinv, b2r, batch3)
    return out16[:, :C]


# speedup vs baseline: 62.5238x; 1.3276x over previous
import functools

import jax
import jax.numpy as jnp
import numpy as np
from jax import lax
from jax.experimental import pallas as pl
from jax.experimental.pallas import tpu as pltpu
from jax.experimental.pallas import tpu_sc as plsc

N = 10000
NP = 10240
E = 320000
D = 128
H16 = 16
C = 10
G = 64

NC = 2
NS = 16
NW = NC * NS
BLK = 128
NBLK = 80
EP = NW * NBLK * BLK
RPT = NP // NS
NBUF = 8
R = 1024
NR = NP // R

_PAD_IDX = np.asarray(N + np.arange(EP - E) % (NP - N), np.int32)
_PAD2 = np.broadcast_to(_PAD_IDX, (2, EP - E))

_mesh = plsc.VectorSubcoreMesh(core_axis_name="c", subcore_axis_name="s")
_sc_params = pltpu.CompilerParams(use_tc_tiling_on_sc=False)
_sc_params_nl = pltpu.CompilerParams(use_tc_tiling_on_sc=False,
                                     needs_layout_passes=False)


@functools.partial(
    pl.kernel,
    out_type=jax.ShapeDtypeStruct((NC, NP), jnp.float32),
    mesh=_mesh,
    scratch_types=[
        pltpu.VMEM((NBLK, BLK), jnp.int32),
        pltpu.VMEM((NP,), jnp.float32),
        pltpu.VMEM((RPT,), jnp.float32),
        pltpu.VMEM((RPT,), jnp.float32),
        pltpu.VMEM_SHARED((NS, NP), jnp.float32),
        pltpu.SemaphoreType.DMA,
    ],
    compiler_params=_sc_params_nl,
)
def _sc_deg(ei_hbm, out_hbm, didx, hist, loc, tmp, stage, sem):
    c = lax.axis_index("c")
    s = lax.axis_index("s")
    wid = c * NS + s
    pltpu.sync_copy(ei_hbm.at[1, pl.ds(wid * NBLK, NBLK)], didx)
    ones = jnp.ones((16,), jnp.float32)
    zero = jnp.zeros((16,), jnp.float32)

    def zloop(i, carry):
        hist[pl.ds(i * 16, 16)] = zero
        return carry
    lax.fori_loop(0, NP // 16, zloop, 0)

    def body(i, carry):
        j = i // 8
        k = (i % 8) * 16
        idx = didx[j, pl.ds(k, 16)]
        plsc.addupdate_scatter(hist, [idx], ones)
        return carry
    lax.fori_loop(0, NBLK * BLK // 16, body, 0)

    pltpu.sync_copy(hist, stage.at[s])
    plsc.subcore_barrier()

    def zl2(i, carry):
        loc[pl.ds(i * 16, 16)] = zero
        return carry
    lax.fori_loop(0, RPT // 16, zl2, 0)

    def red(t, carry):
        pltpu.sync_copy(stage.at[t, pl.ds(s * RPT, RPT)], tmp)

        def add(i, carry2):
            loc[pl.ds(i * 16, 16)] = (loc[pl.ds(i * 16, 16)]
                                      + tmp[pl.ds(i * 16, 16)])
            return carry2
        lax.fori_loop(0, RPT // 16, add, 0)
        return carry
    lax.fori_loop(0, NS, red, 0)
    pltpu.sync_copy(loc, out_hbm.at[c, pl.ds(s * RPT, RPT)])


@functools.partial(
    pl.kernel,
    out_type=jax.ShapeDtypeStruct((NC, NP, H16), jnp.float32),
    mesh=_mesh,
    scratch_types=[
        pltpu.VMEM((NBLK, BLK), jnp.int32),
        pltpu.VMEM((NBLK, BLK), jnp.int32),
        pltpu.VMEM((NBUF, BLK, H16), jnp.float32),
        pltpu.VMEM_SHARED((NP, H16), jnp.float32),
        pltpu.SemaphoreType.DMA((NBUF,)),
        pltpu.SemaphoreType.DMA((NBUF,)),
    ],
    compiler_params=_sc_params,
)
def _sc_agg(g_hbm, ei_hbm, z16_hbm, out_hbm, sidx, didx, rows, acc,
            gsem, ssem):
    c = lax.axis_index("c")
    s = lax.axis_index("s")
    wid = c * NS + s
    pltpu.sync_copy(z16_hbm, acc.at[pl.ds(s * RPT, RPT)])
    pltpu.sync_copy(ei_hbm.at[0, pl.ds(wid * NBLK, NBLK)], sidx)
    pltpu.sync_copy(ei_hbm.at[1, pl.ds(wid * NBLK, NBLK)], didx)
    plsc.subcore_barrier()

    def grp(gi, carry):
        for b in range(NBUF):
            j = gi * NBUF + b

            @pl.when(gi >= 1)
            def _drain():
                pltpu.make_async_copy(rows.at[b], acc.at[didx.at[j]],
                                      ssem.at[b]).wait()

            pltpu.async_copy(g_hbm.at[sidx.at[j]], rows.at[b], gsem.at[b])
        for b in range(NBUF):
            j = gi * NBUF + b
            pltpu.make_async_copy(g_hbm.at[sidx.at[j]], rows.at[b],
                                  gsem.at[b]).wait()
            pltpu.async_copy(rows.at[b], acc.at[didx.at[j]], ssem.at[b],
                             add=True)
        return carry

    lax.fori_loop(0, NBLK // NBUF, grp, 0)
    for b in range(NBUF):
        pltpu.make_async_copy(rows.at[b], acc.at[didx.at[NBLK - NBUF + b]],
                              ssem.at[b]).wait()
    plsc.subcore_barrier()
    pltpu.sync_copy(acc.at[pl.ds(s * RPT, RPT)],
                    out_hbm.at[c, pl.ds(s * RPT, RPT)])


def _tc1a_body(x_ref, w_ref, h_ref):
    h_ref[...] = jnp.dot(x_ref[...], w_ref[...],
                         preferred_element_type=jnp.float32)


def _tc1a(x_pad, W1):
    return pl.pallas_call(
        _tc1a_body,
        grid=(NR,),
        in_specs=[
            pl.BlockSpec((R, D), lambda i: (i, 0)),
            pl.BlockSpec((D, H16), lambda i: (0, 0)),
        ],
        out_specs=pl.BlockSpec((R, H16), lambda i: (i, 0)),
        out_shape=jax.ShapeDtypeStruct((NP, H16), jnp.float32),
    )(x_pad, W1)


def _tc1b_body(deg_ref, h_ref, g_ref, dinv_ref):
    i = pl.program_id(0)
    deg = deg_ref[0] + deg_ref[1] + 1.0
    dinv = lax.rsqrt(jnp.maximum(deg, 1.0))
    rows = lax.iota(jnp.int32, R) + i * R
    dinv = jnp.where(rows < N, dinv, 0.0)
    dinv = dinv[:, None]
    g_ref[...] = dinv * h_ref[...]
    dinv_ref[...] = dinv


def _tc1b(degp, h1):
    return pl.pallas_call(
        _tc1b_body,
        grid=(NR,),
        in_specs=[
            pl.BlockSpec((NC, R), lambda i: (0, i)),
            pl.BlockSpec((R, H16), lambda i: (i, 0)),
        ],
        out_specs=[
            pl.BlockSpec((R, H16), lambda i: (i, 0)),
            pl.BlockSpec((R, 1), lambda i: (i, 0)),
        ],
        out_shape=[
            jax.ShapeDtypeStruct((NP, H16), jnp.float32),
            jax.ShapeDtypeStruct((NP, 1), jnp.float32),
        ],
    )(degp, h1)


def _tc2_body(p_ref, h1_ref, dinv_ref, b1_ref, w2_ref, g2_ref, h2_ref):
    i = pl.program_id(0)
    dinv = dinv_ref[...]
    out1 = dinv * (p_ref[0] + p_ref[1]) \
        + dinv * dinv * h1_ref[...] + b1_ref[...]
    out1 = jnp.maximum(out1, 0.0)
    rows = lax.broadcasted_iota(jnp.int32, (R, 1), 0) + i * R
    out1 = jnp.where(rows < N, out1, 0.0)
    h2 = jnp.dot(out1, w2_ref[...], preferred_element_type=jnp.float32)
    h2_ref[...] = h2
    g2_ref[...] = dinv * h2


def _tc2(p, h1, dinv, b1r, W2p):
    return pl.pallas_call(
        _tc2_body,
        grid=(NR,),
        in_specs=[
            pl.BlockSpec((NC, R, H16), lambda i: (0, i, 0)),
            pl.BlockSpec((R, H16), lambda i: (i, 0)),
            pl.BlockSpec((R, 1), lambda i: (i, 0)),
            pl.BlockSpec((1, H16), lambda i: (0, 0)),
            pl.BlockSpec((H16, H16), lambda i: (0, 0)),
        ],
        out_specs=[
            pl.BlockSpec((R, H16), lambda i: (i, 0)),
            pl.BlockSpec((R, H16), lambda i: (i, 0)),
        ],
        out_shape=[
            jax.ShapeDtypeStruct((NP, H16), jnp.float32),
            jax.ShapeDtypeStruct((NP, H16), jnp.float32),
        ],
    )(p, h1, dinv, b1r, W2p)


def _tc3_body(q_ref, h2_ref, dinv_ref, b2_ref, batch_ref, out_ref, acc):
    i = pl.program_id(0)

    @pl.when(i == 0)
    def _init():
        acc[...] = jnp.zeros((G, H16), jnp.float32)

    dinv = dinv_ref[...]
    out2 = dinv * (q_ref[0] + q_ref[1]) \
        + dinv * dinv * h2_ref[...] + b2_ref[...]
    rows = lax.broadcasted_iota(jnp.int32, (R, 1), 0) + i * R
    m = (rows < N).astype(jnp.float32)
    out2 = out2 * m
    colmask = lax.broadcasted_iota(jnp.int32, (R, H16), 1) == (H16 - 1)
    out2 = jnp.where(colmask, m, out2)
    bb = batch_ref[0, 0, :]
    onehot = (bb[None, :] ==
              lax.broadcasted_iota(jnp.int32, (G, R), 0)).astype(jnp.float32)
    acc[...] += jnp.dot(onehot, out2, preferred_element_type=jnp.float32)

    @pl.when(i == NR - 1)
    def _fin():
        a = acc[...]
        cnt = jnp.maximum(a[:, H16 - 1:H16], 1.0)
        out_ref[...] = jax.nn.sigmoid(a / cnt)


def _tc3(q, h2, dinv, b2r, batch3):
    return pl.pallas_call(
        _tc3_body,
        grid=(NR,),
        in_specs=[
            pl.BlockSpec((NC, R, H16), lambda i: (0, i, 0)),
            pl.BlockSpec((R, H16), lambda i: (i, 0)),
            pl.BlockSpec((R, 1), lambda i: (i, 0)),
            pl.BlockSpec((1, H16), lambda i: (0, 0)),
            pl.BlockSpec((1, 1, R), lambda i: (i, 0, 0)),
        ],
        out_specs=pl.BlockSpec((G, H16), lambda i: (0, 0)),
        out_shape=jax.ShapeDtypeStruct((G, H16), jnp.float32),
        scratch_shapes=[pltpu.VMEM((G, H16), jnp.float32)],
    )(q, h2, dinv, b2r, batch3)


def kernel(x, edge_index, batch, W1, b1, W2, b2):
    f32 = jnp.float32
    ei_r = jnp.concatenate(
        [edge_index, jnp.asarray(_PAD2)], axis=1).reshape(2, NW * NBLK, BLK)
    x_pad = jnp.zeros((NP, D), f32).at[:N].set(x)
    z16 = jnp.zeros((RPT, H16), f32)
    b1r = b1.reshape(1, H16)
    W2p = jnp.zeros((H16, H16), f32).at[:, :C].set(W2)
    b2r = jnp.zeros((1, H16), f32).at[0, :C].set(b2)
    batch3 = jnp.concatenate(
        [batch, jnp.zeros((NP - N,), jnp.int32)]).reshape(NR, 1, R)

    h1 = _tc1a(x_pad, W1)
    degp = _sc_deg(ei_r)
    g1, dinv = _tc1b(degp, h1)
    p = _sc_agg(g1, ei_r, z16)
    g2, h2 = _tc2(p, h1, dinv, b1r, W2p)
    q = _sc_agg(g2, ei_r, z16)
    out16 = _tc3(q, h2, dinv, b2r, batch3)
    return out16[:, :C]


# speedup vs baseline: 65.0646x; 1.0406x over previous
import functools

import jax
import jax.numpy as jnp
import numpy as np
from jax import lax
from jax.experimental import pallas as pl
from jax.experimental.pallas import tpu as pltpu
from jax.experimental.pallas import tpu_sc as plsc

N = 10000
NP = 10240
E = 320000
D = 128
H16 = 16
C = 10
G = 64

NC = 2
NS = 16
NW = NC * NS
BLK = 128
NBLK = 80
EP = NW * NBLK * BLK
RPT = NP // NS
NBUF = 8
R = 2048
NR = NP // R
EW = E // NW

_PAD_IDX = np.asarray(N + np.arange(EP - E) % (NP - N), np.int32)
_PAD2 = np.broadcast_to(_PAD_IDX, (2, EP - E))

_mesh = plsc.VectorSubcoreMesh(core_axis_name="c", subcore_axis_name="s")
_sc_params = pltpu.CompilerParams(use_tc_tiling_on_sc=False)
_sc_params_nl = pltpu.CompilerParams(use_tc_tiling_on_sc=False,
                                     needs_layout_passes=False)


@functools.partial(
    pl.kernel,
    out_type=jax.ShapeDtypeStruct((NC, NP), jnp.float32),
    mesh=_mesh,
    scratch_types=[
        pltpu.VMEM((EW,), jnp.int32),
        pltpu.VMEM((NP,), jnp.float32),
        pltpu.VMEM((RPT,), jnp.float32),
        pltpu.VMEM((RPT,), jnp.float32),
        pltpu.VMEM_SHARED((NS, NP), jnp.float32),
        pltpu.SemaphoreType.DMA,
    ],
    compiler_params=_sc_params_nl,
)
def _sc_deg(ei_flat_hbm, out_hbm, didx, hist, loc, tmp, stage, sem):
    c = lax.axis_index("c")
    s = lax.axis_index("s")
    wid = c * NS + s
    pltpu.sync_copy(ei_flat_hbm.at[pl.ds(E + wid * EW, EW)], didx)
    ones = jnp.ones((16,), jnp.float32)
    zero = jnp.zeros((16,), jnp.float32)

    def zloop(i, carry):
        hist[pl.ds(i * 16, 16)] = zero
        return carry
    lax.fori_loop(0, NP // 16, zloop, 0)

    def body(i, carry):
        idx = didx[pl.ds(i * 16, 16)]
        plsc.addupdate_scatter(hist, [idx], ones)
        return carry
    lax.fori_loop(0, EW // 16, body, 0)

    pltpu.sync_copy(hist, stage.at[s])
    plsc.subcore_barrier()

    def zl2(i, carry):
        loc[pl.ds(i * 16, 16)] = zero
        return carry
    lax.fori_loop(0, RPT // 16, zl2, 0)

    def red(t, carry):
        pltpu.sync_copy(stage.at[t, pl.ds(s * RPT, RPT)], tmp)

        def add(i, carry2):
            loc[pl.ds(i * 16, 16)] = (loc[pl.ds(i * 16, 16)]
                                      + tmp[pl.ds(i * 16, 16)])
            return carry2
        lax.fori_loop(0, RPT // 16, add, 0)
        return carry
    lax.fori_loop(0, NS, red, 0)
    pltpu.sync_copy(loc, out_hbm.at[c, pl.ds(s * RPT, RPT)])


@functools.partial(
    pl.kernel,
    out_type=jax.ShapeDtypeStruct((NC, NP, H16), jnp.float32),
    mesh=_mesh,
    scratch_types=[
        pltpu.VMEM((NBLK, BLK), jnp.int32),
        pltpu.VMEM((NBLK, BLK), jnp.int32),
        pltpu.VMEM((NBUF, BLK, H16), jnp.float32),
        pltpu.VMEM_SHARED((NP, H16), jnp.float32),
        pltpu.SemaphoreType.DMA((NBUF,)),
        pltpu.SemaphoreType.DMA((NBUF,)),
    ],
    compiler_params=_sc_params,
)
def _sc_agg(g_hbm, ei_hbm, z16_hbm, out_hbm, sidx, didx, rows, acc,
            gsem, ssem):
    c = lax.axis_index("c")
    s = lax.axis_index("s")
    wid = c * NS + s
    pltpu.sync_copy(z16_hbm, acc.at[pl.ds(s * RPT, RPT)])
    pltpu.sync_copy(ei_hbm.at[0, pl.ds(wid * NBLK, NBLK)], sidx)
    pltpu.sync_copy(ei_hbm.at[1, pl.ds(wid * NBLK, NBLK)], didx)
    plsc.subcore_barrier()

    def grp(gi, carry):
        for b in range(NBUF):
            j = gi * NBUF + b

            @pl.when(gi >= 1)
            def _drain():
                pltpu.make_async_copy(rows.at[b], acc.at[didx.at[j]],
                                      ssem.at[b]).wait()

            pltpu.async_copy(g_hbm.at[sidx.at[j]], rows.at[b], gsem.at[b])
        for b in range(NBUF):
            j = gi * NBUF + b
            pltpu.make_async_copy(g_hbm.at[sidx.at[j]], rows.at[b],
                                  gsem.at[b]).wait()
            pltpu.async_copy(rows.at[b], acc.at[didx.at[j]], ssem.at[b],
                             add=True)
        return carry

    lax.fori_loop(0, NBLK // NBUF, grp, 0)
    for b in range(NBUF):
        pltpu.make_async_copy(rows.at[b], acc.at[didx.at[NBLK - NBUF + b]],
                              ssem.at[b]).wait()
    plsc.subcore_barrier()
    pltpu.sync_copy(acc.at[pl.ds(s * RPT, RPT)],
                    out_hbm.at[c, pl.ds(s * RPT, RPT)])


def _tc1a_body(x_ref, w_ref, h_ref):
    h_ref[...] = jnp.dot(x_ref[...], w_ref[...],
                         preferred_element_type=jnp.float32)


def _tc1a(x_pad, W1):
    return pl.pallas_call(
        _tc1a_body,
        grid=(NR,),
        in_specs=[
            pl.BlockSpec((R, D), lambda i: (i, 0)),
            pl.BlockSpec((D, H16), lambda i: (0, 0)),
        ],
        out_specs=pl.BlockSpec((R, H16), lambda i: (i, 0)),
        out_shape=jax.ShapeDtypeStruct((NP, H16), jnp.float32),
    )(x_pad, W1)


def _tc1b_body(deg_ref, h_ref, g_ref, dinv_ref):
    i = pl.program_id(0)
    deg = deg_ref[0] + deg_ref[1] + 1.0
    dinv = lax.rsqrt(jnp.maximum(deg, 1.0))
    rows = lax.iota(jnp.int32, R) + i * R
    dinv = jnp.where(rows < N, dinv, 0.0)
    dinv = dinv[:, None]
    g_ref[...] = dinv * h_ref[...]
    dinv_ref[...] = dinv


def _tc1b(degp, h1):
    return pl.pallas_call(
        _tc1b_body,
        grid=(NR,),
        in_specs=[
            pl.BlockSpec((NC, R), lambda i: (0, i)),
            pl.BlockSpec((R, H16), lambda i: (i, 0)),
        ],
        out_specs=[
            pl.BlockSpec((R, H16), lambda i: (i, 0)),
            pl.BlockSpec((R, 1), lambda i: (i, 0)),
        ],
        out_shape=[
            jax.ShapeDtypeStruct((NP, H16), jnp.float32),
            jax.ShapeDtypeStruct((NP, 1), jnp.float32),
        ],
    )(degp, h1)


def _tc2_body(p_ref, h1_ref, dinv_ref, b1_ref, w2_ref, g2_ref, h2_ref):
    i = pl.program_id(0)
    dinv = dinv_ref[...]
    out1 = dinv * (p_ref[0] + p_ref[1]) \
        + dinv * dinv * h1_ref[...] + b1_ref[...]
    out1 = jnp.maximum(out1, 0.0)
    rows = lax.broadcasted_iota(jnp.int32, (R, 1), 0) + i * R
    out1 = jnp.where(rows < N, out1, 0.0)
    h2 = jnp.dot(out1, w2_ref[...], preferred_element_type=jnp.float32)
    h2_ref[...] = h2
    g2_ref[...] = dinv * h2


def _tc2(p, h1, dinv, b1r, W2p):
    return pl.pallas_call(
        _tc2_body,
        grid=(NR,),
        in_specs=[
            pl.BlockSpec((NC, R, H16), lambda i: (0, i, 0)),
            pl.BlockSpec((R, H16), lambda i: (i, 0)),
            pl.BlockSpec((R, 1), lambda i: (i, 0)),
            pl.BlockSpec((1, H16), lambda i: (0, 0)),
            pl.BlockSpec((H16, H16), lambda i: (0, 0)),
        ],
        out_specs=[
            pl.BlockSpec((R, H16), lambda i: (i, 0)),
            pl.BlockSpec((R, H16), lambda i: (i, 0)),
        ],
        out_shape=[
            jax.ShapeDtypeStruct((NP, H16), jnp.float32),
            jax.ShapeDtypeStruct((NP, H16), jnp.float32),
        ],
    )(p, h1, dinv, b1r, W2p)


def _tc3_body(q_ref, h2_ref, dinv_ref, b2_ref, batch_ref, out_ref, acc):
    i = pl.program_id(0)

    @pl.when(i == 0)
    def _init():
        acc[...] = jnp.zeros((G, H16), jnp.float32)

    dinv = dinv_ref[...]
    out2 = dinv * (q_ref[0] + q_ref[1]) \
        + dinv * dinv * h2_ref[...] + b2_ref[...]
    rows = lax.broadcasted_iota(jnp.int32, (R, 1), 0) + i * R
    m = (rows < N).astype(jnp.float32)
    out2 = out2 * m
    colmask = lax.broadcasted_iota(jnp.int32, (R, H16), 1) == (H16 - 1)
    out2 = jnp.where(colmask, m, out2)
    bb = batch_ref[0, 0, :]
    onehot = (bb[None, :] ==
              lax.broadcasted_iota(jnp.int32, (G, R), 0)).astype(jnp.float32)
    acc[...] += jnp.dot(onehot, out2, preferred_element_type=jnp.float32)

    @pl.when(i == NR - 1)
    def _fin():
        a = acc[...]
        cnt = jnp.maximum(a[:, H16 - 1:H16], 1.0)
        out_ref[...] = jax.nn.sigmoid(a / cnt)


def _tc3(q, h2, dinv, b2r, batch3):
    return pl.pallas_call(
        _tc3_body,
        grid=(NR,),
        in_specs=[
            pl.BlockSpec((NC, R, H16), lambda i: (0, i, 0)),
            pl.BlockSpec((R, H16), lambda i: (i, 0)),
            pl.BlockSpec((R, 1), lambda i: (i, 0)),
            pl.BlockSpec((1, H16), lambda i: (0, 0)),
            pl.BlockSpec((1, 1, R), lambda i: (i, 0, 0)),
        ],
        out_specs=pl.BlockSpec((G, H16), lambda i: (0, 0)),
        out_shape=jax.ShapeDtypeStruct((G, H16), jnp.float32),
        scratch_shapes=[pltpu.VMEM((G, H16), jnp.float32)],
    )(q, h2, dinv, b2r, batch3)


def kernel(x, edge_index, batch, W1, b1, W2, b2):
    f32 = jnp.float32
    ei_r = jnp.concatenate(
        [edge_index, jnp.asarray(_PAD2)], axis=1).reshape(2, NW * NBLK, BLK)
    x_pad = jnp.zeros((NP, D), f32).at[:N].set(x)
    z16 = jnp.zeros((RPT, H16), f32)
    b1r = b1.reshape(1, H16)
    W2p = jnp.zeros((H16, H16), f32).at[:, :C].set(W2)
    b2r = jnp.zeros((1, H16), f32).at[0, :C].set(b2)
    batch3 = jnp.concatenate(
        [batch, jnp.zeros((NP - N,), jnp.int32)]).reshape(NR, 1, R)

    h1 = _tc1a(x_pad, W1)
    degp = _sc_deg(edge_index.reshape(2 * E))
    g1, dinv = _tc1b(degp, h1)
    p = _sc_agg(g1, ei_r, z16)
    g2, h2 = _tc2(p, h1, dinv, b1r, W2p)
    q = _sc_agg(g2, ei_r, z16)
    out16 = _tc3(q, h2, dinv, b2r, batch3)
    return out16[:, :C]


# speedup vs baseline: 66.4485x; 1.0213x over previous
import functools

import jax
import jax.numpy as jnp
import numpy as np
from jax import lax
from jax.experimental import pallas as pl
from jax.experimental.pallas import tpu as pltpu
from jax.experimental.pallas import tpu_sc as plsc

N = 10000
NP = 10240
E = 320000
D = 128
H16 = 16
C = 10
G = 64

NC = 2
NS = 16
NW = NC * NS
BLK = 512
NBLK = 20
EP = NW * NBLK * BLK
RPT = NP // NS
NBUF = 4
R = 2048
NR = NP // R
EW = E // NW

_PAD_IDX = np.asarray(N + np.arange(EP - E) % (NP - N), np.int32)
_PAD2 = np.broadcast_to(_PAD_IDX, (2, EP - E))

_mesh = plsc.VectorSubcoreMesh(core_axis_name="c", subcore_axis_name="s")
_sc_params = pltpu.CompilerParams(use_tc_tiling_on_sc=False)
_sc_params_nl = pltpu.CompilerParams(use_tc_tiling_on_sc=False,
                                     needs_layout_passes=False)


@functools.partial(
    pl.kernel,
    out_type=jax.ShapeDtypeStruct((NC, NP), jnp.float32),
    mesh=_mesh,
    scratch_types=[
        pltpu.VMEM((EW,), jnp.int32),
        pltpu.VMEM((NP,), jnp.float32),
        pltpu.VMEM((RPT,), jnp.float32),
        pltpu.VMEM((RPT,), jnp.float32),
        pltpu.VMEM_SHARED((NS, NP), jnp.float32),
        pltpu.SemaphoreType.DMA,
    ],
    compiler_params=_sc_params_nl,
)
def _sc_deg(ei_flat_hbm, out_hbm, didx, hist, loc, tmp, stage, sem):
    c = lax.axis_index("c")
    s = lax.axis_index("s")
    wid = c * NS + s
    pltpu.sync_copy(ei_flat_hbm.at[pl.ds(E + wid * EW, EW)], didx)
    ones = jnp.ones((16,), jnp.float32)
    zero = jnp.zeros((16,), jnp.float32)

    def zloop(i, carry):
        hist[pl.ds(i * 16, 16)] = zero
        return carry
    lax.fori_loop(0, NP // 16, zloop, 0)

    def body(i, carry):
        for u in range(5):
            idx = didx[pl.ds((i * 5 + u) * 16, 16)]
            plsc.addupdate_scatter(hist, [idx], ones)
        return carry
    lax.fori_loop(0, EW // 80, body, 0)

    pltpu.sync_copy(hist, stage.at[s])
    plsc.subcore_barrier()

    def zl2(i, carry):
        loc[pl.ds(i * 16, 16)] = zero
        return carry
    lax.fori_loop(0, RPT // 16, zl2, 0)

    def red(t, carry):
        pltpu.sync_copy(stage.at[t, pl.ds(s * RPT, RPT)], tmp)

        def add(i, carry2):
            loc[pl.ds(i * 16, 16)] = (loc[pl.ds(i * 16, 16)]
                                      + tmp[pl.ds(i * 16, 16)])
            return carry2
        lax.fori_loop(0, RPT // 16, add, 0)
        return carry
    lax.fori_loop(0, NS, red, 0)
    pltpu.sync_copy(loc, out_hbm.at[c, pl.ds(s * RPT, RPT)])


@functools.partial(
    pl.kernel,
    out_type=jax.ShapeDtypeStruct((NC, NP, H16), jnp.float32),
    mesh=_mesh,
    scratch_types=[
        pltpu.VMEM((NBLK, BLK), jnp.int32),
        pltpu.VMEM((NBLK, BLK), jnp.int32),
        pltpu.VMEM((NBUF, BLK, H16), jnp.float32),
        pltpu.VMEM_SHARED((NP, H16), jnp.float32),
        pltpu.SemaphoreType.DMA((NBUF,)),
        pltpu.SemaphoreType.DMA((NBUF,)),
    ],
    compiler_params=_sc_params,
)
def _sc_agg(g_hbm, ei_hbm, z16_hbm, out_hbm, sidx, didx, rows, acc,
            gsem, ssem):
    c = lax.axis_index("c")
    s = lax.axis_index("s")
    wid = c * NS + s
    pltpu.sync_copy(z16_hbm, acc.at[pl.ds(s * RPT, RPT)])
    pltpu.sync_copy(ei_hbm.at[0, pl.ds(wid * NBLK, NBLK)], sidx)
    pltpu.sync_copy(ei_hbm.at[1, pl.ds(wid * NBLK, NBLK)], didx)
    plsc.subcore_barrier()

    def grp(gi, carry):
        for b in range(NBUF):
            j = gi * NBUF + b

            @pl.when(gi >= 1)
            def _drain():
                pltpu.make_async_copy(rows.at[b], acc.at[didx.at[j]],
                                      ssem.at[b]).wait()

            pltpu.async_copy(g_hbm.at[sidx.at[j]], rows.at[b], gsem.at[b])
        for b in range(NBUF):
            j = gi * NBUF + b
            pltpu.make_async_copy(g_hbm.at[sidx.at[j]], rows.at[b],
                                  gsem.at[b]).wait()
            pltpu.async_copy(rows.at[b], acc.at[didx.at[j]], ssem.at[b],
                             add=True)
        return carry

    lax.fori_loop(0, NBLK // NBUF, grp, 0)
    for b in range(NBUF):
        pltpu.make_async_copy(rows.at[b], acc.at[didx.at[NBLK - NBUF + b]],
                              ssem.at[b]).wait()
    plsc.subcore_barrier()
    pltpu.sync_copy(acc.at[pl.ds(s * RPT, RPT)],
                    out_hbm.at[c, pl.ds(s * RPT, RPT)])


def _tc1a_body(x_ref, w_ref, h_ref):
    h_ref[...] = jnp.dot(x_ref[...], w_ref[...],
                         preferred_element_type=jnp.float32)


def _tc1a(x_pad, W1):
    return pl.pallas_call(
        _tc1a_body,
        grid=(NR,),
        in_specs=[
            pl.BlockSpec((R, D), lambda i: (i, 0)),
            pl.BlockSpec((D, H16), lambda i: (0, 0)),
        ],
        out_specs=pl.BlockSpec((R, H16), lambda i: (i, 0)),
        out_shape=jax.ShapeDtypeStruct((NP, H16), jnp.float32),
    )(x_pad, W1)


def _tc1b_body(deg_ref, h_ref, g_ref, dinv_ref):
    i = pl.program_id(0)
    deg = deg_ref[0] + deg_ref[1] + 1.0
    dinv = lax.rsqrt(jnp.maximum(deg, 1.0))
    rows = lax.iota(jnp.int32, R) + i * R
    dinv = jnp.where(rows < N, dinv, 0.0)
    dinv = dinv[:, None]
    g_ref[...] = dinv * h_ref[...]
    dinv_ref[...] = dinv


def _tc1b(degp, h1):
    return pl.pallas_call(
        _tc1b_body,
        grid=(NR,),
        in_specs=[
            pl.BlockSpec((NC, R), lambda i: (0, i)),
            pl.BlockSpec((R, H16), lambda i: (i, 0)),
        ],
        out_specs=[
            pl.BlockSpec((R, H16), lambda i: (i, 0)),
            pl.BlockSpec((R, 1), lambda i: (i, 0)),
        ],
        out_shape=[
            jax.ShapeDtypeStruct((NP, H16), jnp.float32),
            jax.ShapeDtypeStruct((NP, 1), jnp.float32),
        ],
    )(degp, h1)


def _tc2_body(p_ref, h1_ref, dinv_ref, b1_ref, w2_ref, g2_ref, h2_ref):
    i = pl.program_id(0)
    dinv = dinv_ref[...]
    out1 = dinv * (p_ref[0] + p_ref[1]) \
        + dinv * dinv * h1_ref[...] + b1_ref[...]
    out1 = jnp.maximum(out1, 0.0)
    rows = lax.broadcasted_iota(jnp.int32, (R, 1), 0) + i * R
    out1 = jnp.where(rows < N, out1, 0.0)
    h2 = jnp.dot(out1, w2_ref[...], preferred_element_type=jnp.float32)
    h2_ref[...] = h2
    g2_ref[...] = dinv * h2


def _tc2(p, h1, dinv, b1r, W2p):
    return pl.pallas_call(
        _tc2_body,
        grid=(NR,),
        in_specs=[
            pl.BlockSpec((NC, R, H16), lambda i: (0, i, 0)),
            pl.BlockSpec((R, H16), lambda i: (i, 0)),
            pl.BlockSpec((R, 1), lambda i: (i, 0)),
            pl.BlockSpec((1, H16), lambda i: (0, 0)),
            pl.BlockSpec((H16, H16), lambda i: (0, 0)),
        ],
        out_specs=[
            pl.BlockSpec((R, H16), lambda i: (i, 0)),
            pl.BlockSpec((R, H16), lambda i: (i, 0)),
        ],
        out_shape=[
            jax.ShapeDtypeStruct((NP, H16), jnp.float32),
            jax.ShapeDtypeStruct((NP, H16), jnp.float32),
        ],
    )(p, h1, dinv, b1r, W2p)


def _tc3_body(q_ref, h2_ref, dinv_ref, b2_ref, batch_ref, out_ref, acc):
    i = pl.program_id(0)

    @pl.when(i == 0)
    def _init():
        acc[...] = jnp.zeros((G, H16), jnp.float32)

    dinv = dinv_ref[...]
    out2 = dinv * (q_ref[0] + q_ref[1]) \
        + dinv * dinv * h2_ref[...] + b2_ref[...]
    rows = lax.broadcasted_iota(jnp.int32, (R, 1), 0) + i * R
    m = (rows < N).astype(jnp.float32)
    out2 = out2 * m
    colmask = lax.broadcasted_iota(jnp.int32, (R, H16), 1) == (H16 - 1)
    out2 = jnp.where(colmask, m, out2)
    bb = batch_ref[0, 0, :]
    onehot = (bb[None, :] ==
              lax.broadcasted_iota(jnp.int32, (G, R), 0)).astype(jnp.float32)
    acc[...] += jnp.dot(onehot, out2, preferred_element_type=jnp.float32)

    @pl.when(i == NR - 1)
    def _fin():
        a = acc[...]
        cnt = jnp.maximum(a[:, H16 - 1:H16], 1.0)
        out_ref[...] = jax.nn.sigmoid(a / cnt)


def _tc3(q, h2, dinv, b2r, batch3):
    return pl.pallas_call(
        _tc3_body,
        grid=(NR,),
        in_specs=[
            pl.BlockSpec((NC, R, H16), lambda i: (0, i, 0)),
            pl.BlockSpec((R, H16), lambda i: (i, 0)),
            pl.BlockSpec((R, 1), lambda i: (i, 0)),
            pl.BlockSpec((1, H16), lambda i: (0, 0)),
            pl.BlockSpec((1, 1, R), lambda i: (i, 0, 0)),
        ],
        out_specs=pl.BlockSpec((G, H16), lambda i: (0, 0)),
        out_shape=jax.ShapeDtypeStruct((G, H16), jnp.float32),
        scratch_shapes=[pltpu.VMEM((G, H16), jnp.float32)],
    )(q, h2, dinv, b2r, batch3)


def kernel(x, edge_index, batch, W1, b1, W2, b2):
    f32 = jnp.float32
    ei_r = jnp.concatenate(
        [edge_index, jnp.asarray(_PAD2)], axis=1).reshape(2, NW * NBLK, BLK)
    x_pad = jnp.zeros((NP, D), f32).at[:N].set(x)
    z16 = jnp.zeros((RPT, H16), f32)
    b1r = b1.reshape(1, H16)
    W2p = jnp.zeros((H16, H16), f32).at[:, :C].set(W2)
    b2r = jnp.zeros((1, H16), f32).at[0, :C].set(b2)
    batch3 = jnp.concatenate(
        [batch, jnp.zeros((NP - N,), jnp.int32)]).reshape(NR, 1, R)

    h1 = _tc1a(x_pad, W1)
    degp = _sc_deg(edge_index.reshape(2 * E))
    g1, dinv = _tc1b(degp, h1)
    p = _sc_agg(g1, ei_r, z16)
    g2, h2 = _tc2(p, h1, dinv, b1r, W2p)
    q = _sc_agg(g2, ei_r, z16)
    out16 = _tc3(q, h2, dinv, b2r, batch3)
    return out16[:, :C]
